# Initial kernel scaffold; baseline (speedup 1.0000x reference)
import jax
import jax.numpy as jnp
from jax.experimental import pallas as pl


def kernel(x, edge_index, W1, b1, W2, b2, W3, b3, Wl, bl):
    raise NotImplementedError("write your pallas kernel here")


# speedup vs baseline: 5.8929x; 5.8929x over previous
import functools

import jax
import jax.numpy as jnp
from jax import lax
from jax.experimental import pallas as pl
from jax.experimental.pallas import tpu as pltpu
from jax.experimental.pallas import tpu_sc as plsc

N = 10000
D = 128
C = 64
E = 320000

NC = 2
NS = 16
NW = NC * NS

N_PAD = 10240
K = 128
EPT = 10240
E_PAD = EPT * NW
CHUNKS = EPT // K
ROWS_PT = N_PAD // NS
RB = ROWS_PT // K

@functools.cache
def _build_sc_kernels():
    mesh = plsc.VectorSubcoreMesh(
        core_axis_name="c", subcore_axis_name="s",
        num_cores=NC, num_subcores=NS)
    sc_deg = functools.partial(
        pl.kernel,
        out_type=jax.ShapeDtypeStruct((NC, N_PAD, 16), jnp.float32),
        mesh=mesh,
        scratch_types=[
            pltpu.VMEM((K,), jnp.int32),
            pltpu.VMEM((K, 16), jnp.float32),
            pltpu.VMEM((K, 16), jnp.float32),
            pltpu.VMEM_SHARED((N_PAD, 16), jnp.float32),
        ],
    )(_sc_deg_body)
    sc_scatter = functools.partial(
        pl.kernel,
        out_type=jax.ShapeDtypeStruct((NC, N_PAD, D), jnp.float32),
        mesh=mesh,
        scratch_types=[
            pltpu.VMEM((K,), jnp.int32),
            pltpu.VMEM((K,), jnp.int32),
            pltpu.VMEM((K, D), jnp.float32),
            pltpu.VMEM_SHARED((N_PAD, D), jnp.float32),
            pltpu.SemaphoreType.DMA,
        ],
    )(_sc_scatter_body)
    return sc_deg, sc_scatter


def _sc_deg_body(dst_hbm, out_hbm, didx, ones_v, stage_v, acc):
    cid = lax.axis_index("c")
    sid = lax.axis_index("s")

    def fill(i, _):
        ones_v[i] = jnp.full((16,), 1.0, jnp.float32)
        stage_v[i] = jnp.zeros((16,), jnp.float32)
        return 0

    lax.fori_loop(0, K, fill, 0)
    r0 = sid * ROWS_PT
    for b in range(RB):
        pltpu.sync_copy(stage_v, acc.at[pl.ds(r0 + b * K, K)])
    plsc.subcore_barrier()

    ebase = (cid * NS + sid) * EPT

    def body(i, _):
        pltpu.sync_copy(dst_hbm.at[pl.ds(ebase + i * K, K)], didx)
        pltpu.sync_copy(ones_v, acc.at[didx], add=True)
        return 0

    lax.fori_loop(0, CHUNKS, body, 0)
    plsc.subcore_barrier()
    for b in range(RB):
        pltpu.sync_copy(acc.at[pl.ds(r0 + b * K, K)], stage_v)
        pltpu.sync_copy(stage_v, out_hbm.at[cid, pl.ds(r0 + b * K, K), :])


def _sc_scatter_body(xp_hbm, src_hbm, dst_hbm, out_hbm, sidx, didx, rows, acc, sem):
    cid = lax.axis_index("c")
    sid = lax.axis_index("s")

    def zero(i, _):
        rows[i // 8, pl.ds((i % 8) * 16, 16)] = jnp.zeros((16,), jnp.float32)
        return 0

    lax.fori_loop(0, K * 8, zero, 0)
    r0 = sid * ROWS_PT
    for b in range(RB):
        pltpu.sync_copy(rows, acc.at[pl.ds(r0 + b * K, K)])
    plsc.subcore_barrier()

    ebase = (cid * NS + sid) * EPT

    def body(i, _):
        base = ebase + i * K
        pltpu.sync_copy(src_hbm.at[pl.ds(base, K)], sidx)
        pltpu.sync_copy(dst_hbm.at[pl.ds(base, K)], didx)
        pltpu.async_copy(xp_hbm.at[sidx], rows, sem).wait()
        pltpu.sync_copy(rows, acc.at[didx], add=True)
        return 0

    lax.fori_loop(0, CHUNKS, body, 0)
    plsc.subcore_barrier()
    for b in range(RB):
        pltpu.sync_copy(acc.at[pl.ds(r0 + b * K, K)], rows)
        pltpu.sync_copy(rows, out_hbm.at[cid, pl.ds(r0 + b * K, K), :])


_BT = 256
_GRID = N_PAD // _BT


def _k1_body(deg_ref, x_ref, dis_ref, xp0_ref):
    deg = jnp.sum(deg_ref[...], axis=(0, 2)) + 1.0
    dis = lax.rsqrt(deg)[:, None]
    dis_ref[...] = dis
    xp0_ref[...] = x_ref[...] * dis


def _mid_body(dis_ref, s_ref, xp_ref, xres_ref, w_ref, b_ref, wn_ref,
              xo_ref, xpn_ref, *, first):
    dis = dis_ref[...]
    t = (s_ref[0] + s_ref[1] + xp_ref[...]) * dis
    if first:
        h = jnp.dot(t, w_ref[...], preferred_element_type=jnp.float32)
        xo = jnp.maximum(h + b_ref[...], 0.0)
    else:
        xo = jnp.maximum(xres_ref[...] + t + b_ref[...], 0.0)
    xo_ref[...] = xo
    xpn_ref[...] = jnp.dot(xo, wn_ref[...], preferred_element_type=jnp.float32) * dis


def _k7_body(dis_ref, s_ref, xp_ref, x2_ref, b3_ref, wl_ref, bl_ref, out_ref):
    dis = dis_ref[...]
    t = (s_ref[0] + s_ref[1] + xp_ref[...]) * dis
    x3 = jnp.maximum(x2_ref[...] + t + b3_ref[...], 0.0)
    o = jnp.dot(x3, wl_ref[...], preferred_element_type=jnp.float32) + bl_ref[...]
    m = jnp.max(o, axis=1, keepdims=True)
    lse = jnp.log(jnp.sum(jnp.exp(o - m), axis=1, keepdims=True)) + m
    out_ref[...] = o - lse


def _row_spec(width):
    return pl.BlockSpec((_BT, width), lambda i: (i, 0))


def _full_spec(shape):
    nd = len(shape)
    return pl.BlockSpec(shape, lambda i: (0,) * nd)


_S_SPEC = pl.BlockSpec((NC, _BT, D), lambda i: (0, i, 0))

_k1 = pl.pallas_call(
    _k1_body,
    grid=(_GRID,),
    in_specs=[pl.BlockSpec((NC, _BT, 16), lambda i: (0, i, 0)), _row_spec(D)],
    out_specs=[_row_spec(1), _row_spec(D)],
    out_shape=[
        jax.ShapeDtypeStruct((N_PAD, 1), jnp.float32),
        jax.ShapeDtypeStruct((N_PAD, D), jnp.float32),
    ],
)

_mid_specs = dict(
    grid=(_GRID,),
    in_specs=[
        _row_spec(1), _S_SPEC, _row_spec(D), _row_spec(D),
        _full_spec((D, D)), _full_spec((1, D)), _full_spec((D, D)),
    ],
    out_specs=[_row_spec(D), _row_spec(D)],
    out_shape=[
        jax.ShapeDtypeStruct((N_PAD, D), jnp.float32),
        jax.ShapeDtypeStruct((N_PAD, D), jnp.float32),
    ],
)
_k3 = pl.pallas_call(functools.partial(_mid_body, first=True), **_mid_specs)
_k5 = pl.pallas_call(functools.partial(_mid_body, first=False), **_mid_specs)

_B7 = 80
_k7 = pl.pallas_call(
    _k7_body,
    grid=(N // _B7,),
    in_specs=[
        pl.BlockSpec((_B7, 1), lambda i: (i, 0)),
        pl.BlockSpec((NC, _B7, D), lambda i: (0, i, 0)),
        pl.BlockSpec((_B7, D), lambda i: (i, 0)),
        pl.BlockSpec((_B7, D), lambda i: (i, 0)),
        _full_spec((1, D)), _full_spec((D, C)), _full_spec((1, C)),
    ],
    out_specs=pl.BlockSpec((_B7, C), lambda i: (i, 0)),
    out_shape=jax.ShapeDtypeStruct((N, C), jnp.float32),
)


def kernel(x, edge_index, W1, b1, W2, b2, W3, b3, Wl, bl):
    src = edge_index[0]
    dst = edge_index[1]
    pad = jnp.full((E_PAD - E,), N, jnp.int32)
    src_p = jnp.concatenate([src, pad])
    dst_p = jnp.concatenate([dst, pad])
    x_p = jnp.pad(x, ((0, N_PAD - N), (0, 0)))

    _sc_deg, _sc_scatter = _build_sc_kernels()
    deg16 = _sc_deg(dst_p)
    dis, xp0 = _k1(deg16, x_p)

    s1 = _sc_scatter(xp0, src_p, dst_p)
    x1, xp1 = _k3(dis, s1, xp0, xp0, W1, b1.reshape(1, D), W2)

    s2 = _sc_scatter(xp1, src_p, dst_p)
    x2, xp2 = _k5(dis, s2, xp1, x1, W1, b2.reshape(1, D), W3)

    s3 = _sc_scatter(xp2, src_p, dst_p)
    out = _k7(dis, s3, xp2, x2, b3.reshape(1, D), Wl, bl.reshape(1, C))
    return out


# speedup vs baseline: 6.9089x; 1.1724x over previous
import functools

import jax
import jax.numpy as jnp
from jax import lax
from jax.experimental import pallas as pl
from jax.experimental.pallas import tpu as pltpu
from jax.experimental.pallas import tpu_sc as plsc

N = 10000
D = 128
C = 64
E = 320000

NC = 2
NS = 16
NW = NC * NS

N_PAD = 10240
K = 128
EPT = 10240
E_PAD = EPT * NW
CHUNKS = EPT // K
ROWS_PT = N_PAD // NS
RB = ROWS_PT // K

@functools.cache
def _build_sc_kernels():
    mesh = plsc.VectorSubcoreMesh(
        core_axis_name="c", subcore_axis_name="s",
        num_cores=NC, num_subcores=NS)
    sc_deg = functools.partial(
        pl.kernel,
        out_type=jax.ShapeDtypeStruct((NC, N_PAD, 16), jnp.float32),
        mesh=mesh,
        scratch_types=[
            pltpu.VMEM((K,), jnp.int32),
            pltpu.VMEM((K,), jnp.int32),
            pltpu.VMEM((K, 16), jnp.float32),
            pltpu.VMEM((K, 16), jnp.float32),
            pltpu.VMEM_SHARED((N_PAD, 16), jnp.float32),
            pltpu.SemaphoreType.DMA,
            pltpu.SemaphoreType.DMA,
        ],
    )(_sc_deg_body)
    sc_scatter = functools.partial(
        pl.kernel,
        out_type=jax.ShapeDtypeStruct((NC, N_PAD, D), jnp.float32),
        mesh=mesh,
        scratch_types=[
            pltpu.VMEM((CHUNKS, K), jnp.int32),
            pltpu.VMEM((K,), jnp.int32),
            pltpu.VMEM((K,), jnp.int32),
            pltpu.VMEM((K, D), jnp.float32),
            pltpu.VMEM((K, D), jnp.float32),
            pltpu.VMEM_SHARED((N_PAD, D), jnp.float32),
            pltpu.SemaphoreType.DMA,
            pltpu.SemaphoreType.DMA,
        ],
    )(_sc_scatter_body)
    return sc_deg, sc_scatter


def _sc_deg_body(dst_hbm, out_hbm, didx0, didx1, ones_v, stage_v, acc, sem0, sem1):
    cid = lax.axis_index("c")
    sid = lax.axis_index("s")
    crow0 = (cid * NS + sid) * CHUNKS

    def fill(i, _):
        ones_v[i] = jnp.full((16,), 1.0, jnp.float32)
        stage_v[i] = jnp.zeros((16,), jnp.float32)
        return 0

    lax.fori_loop(0, K, fill, 0)
    r0 = sid * ROWS_PT
    for b in range(RB):
        pltpu.sync_copy(stage_v, acc.at[pl.ds(r0 + b * K, K)])
    pltpu.async_copy(dst_hbm.at[crow0], didx0, sem0)
    pltpu.async_copy(dst_hbm.at[crow0 + 1], didx1, sem1)
    plsc.subcore_barrier()

    bufs = ((didx0, sem0), (didx1, sem1))

    def body(j, _):
        for b in range(2):
            didx, sem = bufs[b]
            i = 2 * j + b
            pltpu.make_async_copy(dst_hbm.at[crow0 + i], didx, sem).wait()
            pltpu.sync_copy(ones_v, acc.at[didx], add=True)

            @pl.when(i + 2 < CHUNKS)
            def _():
                pltpu.async_copy(dst_hbm.at[crow0 + i + 2], didx, sem)
        return 0

    lax.fori_loop(0, CHUNKS // 2, body, 0)
    plsc.subcore_barrier()
    for b in range(RB):
        pltpu.sync_copy(acc.at[pl.ds(r0 + b * K, K)], stage_v)
        pltpu.sync_copy(stage_v, out_hbm.at[cid, pl.ds(r0 + b * K, K), :])


def _sc_scatter_body(xp_hbm, src_hbm, dst_hbm, out_hbm,
                     sall, didx0, didx1, rows0, rows1, acc, sem0, sem1):
    cid = lax.axis_index("c")
    sid = lax.axis_index("s")
    crow0 = (cid * NS + sid) * CHUNKS

    def zero(i, _):
        rows0[i // 8, pl.ds((i % 8) * 16, 16)] = jnp.zeros((16,), jnp.float32)
        return 0

    lax.fori_loop(0, K * 8, zero, 0)
    r0 = sid * ROWS_PT
    for b in range(RB):
        pltpu.sync_copy(rows0, acc.at[pl.ds(r0 + b * K, K)])
    pltpu.sync_copy(src_hbm.at[pl.ds(crow0, CHUNKS)], sall)
    pltpu.async_copy(dst_hbm.at[crow0], didx0, sem0)
    pltpu.async_copy(xp_hbm.at[sall.at[0]], rows0, sem0)
    pltpu.async_copy(dst_hbm.at[crow0 + 1], didx1, sem1)
    pltpu.async_copy(xp_hbm.at[sall.at[1]], rows1, sem1)
    plsc.subcore_barrier()

    bufs = ((didx0, rows0, sem0), (didx1, rows1, sem1))

    def body(j, _):
        for b in range(2):
            didx, rows, sem = bufs[b]
            i = 2 * j + b
            pltpu.make_async_copy(dst_hbm.at[crow0 + i], didx, sem).wait()
            pltpu.make_async_copy(xp_hbm.at[sall.at[i]], rows, sem).wait()
            pltpu.sync_copy(rows, acc.at[didx], add=True)

            @pl.when(i + 2 < CHUNKS)
            def _():
                pltpu.async_copy(dst_hbm.at[crow0 + i + 2], didx, sem)
                pltpu.async_copy(xp_hbm.at[sall.at[i + 2]], rows, sem)
        return 0

    lax.fori_loop(0, CHUNKS // 2, body, 0)
    plsc.subcore_barrier()
    for b in range(RB):
        pltpu.sync_copy(acc.at[pl.ds(r0 + b * K, K)], rows0)
        pltpu.sync_copy(rows0, out_hbm.at[cid, pl.ds(r0 + b * K, K), :])


_BT = 256
_GRID = N_PAD // _BT


def _k1_body(deg_ref, x_ref, dis_ref, xp0_ref):
    deg = jnp.sum(deg_ref[...], axis=(0, 2)) + 1.0
    dis = lax.rsqrt(deg)[:, None]
    dis_ref[...] = dis
    xp0_ref[...] = x_ref[...] * dis


def _mid_body(dis_ref, s_ref, xp_ref, xres_ref, w_ref, b_ref, wn_ref,
              xo_ref, xpn_ref, *, first):
    dis = dis_ref[...]
    t = (s_ref[0] + s_ref[1] + xp_ref[...]) * dis
    if first:
        h = jnp.dot(t, w_ref[...], preferred_element_type=jnp.float32)
        xo = jnp.maximum(h + b_ref[...], 0.0)
    else:
        xo = jnp.maximum(xres_ref[...] + t + b_ref[...], 0.0)
    xo_ref[...] = xo
    xpn_ref[...] = jnp.dot(xo, wn_ref[...], preferred_element_type=jnp.float32) * dis


def _k7_body(dis_ref, s_ref, xp_ref, x2_ref, b3_ref, wl_ref, bl_ref, out_ref):
    dis = dis_ref[...]
    t = (s_ref[0] + s_ref[1] + xp_ref[...]) * dis
    x3 = jnp.maximum(x2_ref[...] + t + b3_ref[...], 0.0)
    o = jnp.dot(x3, wl_ref[...], preferred_element_type=jnp.float32) + bl_ref[...]
    m = jnp.max(o, axis=1, keepdims=True)
    lse = jnp.log(jnp.sum(jnp.exp(o - m), axis=1, keepdims=True)) + m
    out_ref[...] = o - lse


def _row_spec(width):
    return pl.BlockSpec((_BT, width), lambda i: (i, 0))


def _full_spec(shape):
    nd = len(shape)
    return pl.BlockSpec(shape, lambda i: (0,) * nd)


_S_SPEC = pl.BlockSpec((NC, _BT, D), lambda i: (0, i, 0))

_k1 = pl.pallas_call(
    _k1_body,
    grid=(_GRID,),
    in_specs=[pl.BlockSpec((NC, _BT, 16), lambda i: (0, i, 0)), _row_spec(D)],
    out_specs=[_row_spec(1), _row_spec(D)],
    out_shape=[
        jax.ShapeDtypeStruct((N_PAD, 1), jnp.float32),
        jax.ShapeDtypeStruct((N_PAD, D), jnp.float32),
    ],
)

_mid_specs = dict(
    grid=(_GRID,),
    in_specs=[
        _row_spec(1), _S_SPEC, _row_spec(D), _row_spec(D),
        _full_spec((D, D)), _full_spec((1, D)), _full_spec((D, D)),
    ],
    out_specs=[_row_spec(D), _row_spec(D)],
    out_shape=[
        jax.ShapeDtypeStruct((N_PAD, D), jnp.float32),
        jax.ShapeDtypeStruct((N_PAD, D), jnp.float32),
    ],
)
_k3 = pl.pallas_call(functools.partial(_mid_body, first=True), **_mid_specs)
_k5 = pl.pallas_call(functools.partial(_mid_body, first=False), **_mid_specs)

_B7 = 80
_k7 = pl.pallas_call(
    _k7_body,
    grid=(N // _B7,),
    in_specs=[
        pl.BlockSpec((_B7, 1), lambda i: (i, 0)),
        pl.BlockSpec((NC, _B7, D), lambda i: (0, i, 0)),
        pl.BlockSpec((_B7, D), lambda i: (i, 0)),
        pl.BlockSpec((_B7, D), lambda i: (i, 0)),
        _full_spec((1, D)), _full_spec((D, C)), _full_spec((1, C)),
    ],
    out_specs=pl.BlockSpec((_B7, C), lambda i: (i, 0)),
    out_shape=jax.ShapeDtypeStruct((N, C), jnp.float32),
)


def kernel(x, edge_index, W1, b1, W2, b2, W3, b3, Wl, bl):
    src = edge_index[0]
    dst = edge_index[1]
    pad = jnp.full((E_PAD - E,), N, jnp.int32)
    src_p = jnp.concatenate([src, pad]).reshape(E_PAD // K, K)
    dst_p = jnp.concatenate([dst, pad]).reshape(E_PAD // K, K)
    x_p = jnp.pad(x, ((0, N_PAD - N), (0, 0)))

    _sc_deg, _sc_scatter = _build_sc_kernels()
    deg16 = _sc_deg(dst_p)
    dis, xp0 = _k1(deg16, x_p)

    s1 = _sc_scatter(xp0, src_p, dst_p)
    x1, xp1 = _k3(dis, s1, xp0, xp0, W1, b1.reshape(1, D), W2)

    s2 = _sc_scatter(xp1, src_p, dst_p)
    x2, xp2 = _k5(dis, s2, xp1, x1, W1, b2.reshape(1, D), W3)

    s3 = _sc_scatter(xp2, src_p, dst_p)
    out = _k7(dis, s3, xp2, x2, b3.reshape(1, D), Wl, bl.reshape(1, C))
    return out


# speedup vs baseline: 8.0382x; 1.1635x over previous
import functools

import jax
import jax.numpy as jnp
from jax import lax
from jax.experimental import pallas as pl
from jax.experimental.pallas import tpu as pltpu
from jax.experimental.pallas import tpu_sc as plsc

N = 10000
D = 128
C = 64
E = 320000

NC = 2
NS = 16
NW = NC * NS

N_PAD = 10240
K = 128
EPT = 10240
E_PAD = EPT * NW
CHUNKS = EPT // K
ROWS_PT = N_PAD // NS
RB = ROWS_PT // K

@functools.cache
def _build_sc_kernels():
    mesh = plsc.VectorSubcoreMesh(
        core_axis_name="c", subcore_axis_name="s",
        num_cores=NC, num_subcores=NS)
    sc_deg = functools.partial(
        pl.kernel,
        out_type=jax.ShapeDtypeStruct((NC, N_PAD, 16), jnp.float32),
        mesh=mesh,
        scratch_types=[
            pltpu.VMEM((K,), jnp.int32),
            pltpu.VMEM((K,), jnp.int32),
            pltpu.VMEM((K, 16), jnp.float32),
            pltpu.VMEM((K, 16), jnp.float32),
            pltpu.VMEM_SHARED((N_PAD, 16), jnp.float32),
            pltpu.SemaphoreType.DMA,
            pltpu.SemaphoreType.DMA,
        ],
    )(_sc_deg_body)
    sc_scatter = functools.partial(
        pl.kernel,
        out_type=jax.ShapeDtypeStruct((NC, N_PAD, D), jnp.float32),
        mesh=mesh,
        scratch_types=[
            pltpu.VMEM((CHUNKS, K), jnp.int32),
            pltpu.VMEM((K,), jnp.int32),
            pltpu.VMEM((K,), jnp.int32),
            pltpu.VMEM((K, D), jnp.float32),
            pltpu.VMEM((K, D), jnp.float32),
            pltpu.VMEM_SHARED((N_PAD, D), jnp.float32),
            pltpu.SemaphoreType.DMA,
            pltpu.SemaphoreType.DMA,
        ],
    )(_sc_scatter_body)
    return sc_deg, sc_scatter


def _sc_deg_body(dst_hbm, out_hbm, didx0, didx1, ones_v, stage_v, acc, sem0, sem1):
    cid = lax.axis_index("c")
    sid = lax.axis_index("s")
    crow0 = (cid * NS + sid) * CHUNKS

    def fill(i, _):
        ones_v[i] = jnp.full((16,), 1.0, jnp.float32)
        stage_v[i] = jnp.zeros((16,), jnp.float32)
        return 0

    lax.fori_loop(0, K, fill, 0)
    r0 = sid * ROWS_PT
    for b in range(RB):
        pltpu.sync_copy(stage_v, acc.at[pl.ds(r0 + b * K, K)])
    pltpu.async_copy(dst_hbm.at[crow0], didx0, sem0)
    pltpu.async_copy(dst_hbm.at[crow0 + 1], didx1, sem1)
    plsc.subcore_barrier()

    bufs = ((didx0, sem0), (didx1, sem1))

    def body(j, _):
        for b in range(2):
            didx, sem = bufs[b]
            i = 2 * j + b
            pltpu.make_async_copy(dst_hbm.at[crow0 + i], didx, sem).wait()
            pltpu.sync_copy(ones_v, acc.at[didx], add=True)

            @pl.when(i + 2 < CHUNKS)
            def _():
                pltpu.async_copy(dst_hbm.at[crow0 + i + 2], didx, sem)
        return 0

    lax.fori_loop(0, CHUNKS // 2, body, 0)
    plsc.subcore_barrier()
    for b in range(RB):
        pltpu.sync_copy(acc.at[pl.ds(r0 + b * K, K)], stage_v)
        pltpu.sync_copy(stage_v, out_hbm.at[cid, pl.ds(r0 + b * K, K), :])


def _sc_scatter_body(xp_hbm, src_hbm, dst_hbm, out_hbm,
                     sall, didx0, didx1, rows0, rows1, acc, sem0, sem1):
    cid = lax.axis_index("c")
    sid = lax.axis_index("s")
    crow0 = (cid * NS + sid) * CHUNKS
    tbl = xp_hbm.at[cid]

    def zero(i, _):
        rows0[i // 8, pl.ds((i % 8) * 16, 16)] = jnp.zeros((16,), jnp.float32)
        return 0

    lax.fori_loop(0, K * 8, zero, 0)
    r0 = sid * ROWS_PT
    for b in range(RB):
        pltpu.sync_copy(rows0, acc.at[pl.ds(r0 + b * K, K)])
    pltpu.sync_copy(src_hbm.at[pl.ds(crow0, CHUNKS)], sall)
    pltpu.async_copy(dst_hbm.at[crow0], didx0, sem0)
    pltpu.async_copy(tbl.at[sall.at[0]], rows0, sem0)
    pltpu.async_copy(dst_hbm.at[crow0 + 1], didx1, sem1)
    pltpu.async_copy(tbl.at[sall.at[1]], rows1, sem1)
    plsc.subcore_barrier()

    bufs = ((didx0, rows0, sem0), (didx1, rows1, sem1))

    def body(j, _):
        for b in range(2):
            didx, rows, sem = bufs[b]
            i = 2 * j + b
            pltpu.make_async_copy(dst_hbm.at[crow0 + i], didx, sem).wait()
            pltpu.make_async_copy(tbl.at[sall.at[i]], rows, sem).wait()
            pltpu.sync_copy(rows, acc.at[didx], add=True)

            @pl.when(i + 2 < CHUNKS)
            def _():
                pltpu.async_copy(dst_hbm.at[crow0 + i + 2], didx, sem)
                pltpu.async_copy(tbl.at[sall.at[i + 2]], rows, sem)
        return 0

    lax.fori_loop(0, CHUNKS // 2, body, 0)
    plsc.subcore_barrier()
    for b in range(RB):
        pltpu.sync_copy(acc.at[pl.ds(r0 + b * K, K)], rows0)
        pltpu.sync_copy(rows0, out_hbm.at[cid, pl.ds(r0 + b * K, K), :])


_BT = 256
_GRID = N_PAD // _BT


def _k1_body(deg_ref, x_ref, dis_ref, xp0_ref):
    deg = jnp.sum(deg_ref[...], axis=(0, 2)) + 1.0
    dis = lax.rsqrt(deg)[:, None]
    dis_ref[...] = dis
    xp0_ref[...] = x_ref[...] * dis


def _mid_body(dis_ref, s_ref, xp_ref, xres_ref, w_ref, b_ref, wn_ref,
              xo_ref, xpn_ref, *, first):
    dis = dis_ref[...]
    t = (s_ref[0] + s_ref[1] + xp_ref[...]) * dis
    if first:
        h = jnp.dot(t, w_ref[...], preferred_element_type=jnp.float32)
        xo = jnp.maximum(h + b_ref[...], 0.0)
    else:
        xo = jnp.maximum(xres_ref[...] + t + b_ref[...], 0.0)
    xo_ref[...] = xo
    xpn_ref[...] = jnp.dot(xo, wn_ref[...], preferred_element_type=jnp.float32) * dis


def _k7_body(dis_ref, s_ref, xp_ref, x2_ref, b3_ref, wl_ref, bl_ref, out_ref):
    dis = dis_ref[...]
    t = (s_ref[0] + s_ref[1] + xp_ref[...]) * dis
    x3 = jnp.maximum(x2_ref[...] + t + b3_ref[...], 0.0)
    o = jnp.dot(x3, wl_ref[...], preferred_element_type=jnp.float32) + bl_ref[...]
    m = jnp.max(o, axis=1, keepdims=True)
    lse = jnp.log(jnp.sum(jnp.exp(o - m), axis=1, keepdims=True)) + m
    out_ref[...] = o - lse


def _row_spec(width):
    return pl.BlockSpec((_BT, width), lambda i: (i, 0))


def _full_spec(shape):
    nd = len(shape)
    return pl.BlockSpec(shape, lambda i: (0,) * nd)


_S_SPEC = pl.BlockSpec((NC, _BT, D), lambda i: (0, i, 0))

_k1 = pl.pallas_call(
    _k1_body,
    grid=(_GRID,),
    in_specs=[pl.BlockSpec((NC, _BT, 16), lambda i: (0, i, 0)), _row_spec(D)],
    out_specs=[_row_spec(1), _row_spec(D)],
    out_shape=[
        jax.ShapeDtypeStruct((N_PAD, 1), jnp.float32),
        jax.ShapeDtypeStruct((N_PAD, D), jnp.float32),
    ],
)

_mid_specs = dict(
    grid=(_GRID,),
    in_specs=[
        _row_spec(1), _S_SPEC, _row_spec(D), _row_spec(D),
        _full_spec((D, D)), _full_spec((1, D)), _full_spec((D, D)),
    ],
    out_specs=[_row_spec(D), _row_spec(D)],
    out_shape=[
        jax.ShapeDtypeStruct((N_PAD, D), jnp.float32),
        jax.ShapeDtypeStruct((N_PAD, D), jnp.float32),
    ],
)
_k3 = pl.pallas_call(functools.partial(_mid_body, first=True), **_mid_specs)
_k5 = pl.pallas_call(functools.partial(_mid_body, first=False), **_mid_specs)

_B7 = 80
_k7 = pl.pallas_call(
    _k7_body,
    grid=(N // _B7,),
    in_specs=[
        pl.BlockSpec((_B7, 1), lambda i: (i, 0)),
        pl.BlockSpec((NC, _B7, D), lambda i: (0, i, 0)),
        pl.BlockSpec((_B7, D), lambda i: (i, 0)),
        pl.BlockSpec((_B7, D), lambda i: (i, 0)),
        _full_spec((1, D)), _full_spec((D, C)), _full_spec((1, C)),
    ],
    out_specs=pl.BlockSpec((_B7, C), lambda i: (i, 0)),
    out_shape=jax.ShapeDtypeStruct((N, C), jnp.float32),
)


def kernel(x, edge_index, W1, b1, W2, b2, W3, b3, Wl, bl):
    src = edge_index[0]
    dst = edge_index[1]
    pad = jnp.full((E_PAD - E,), N, jnp.int32)
    src_p = jnp.concatenate([src, pad]).reshape(E_PAD // K, K)
    dst_p = jnp.concatenate([dst, pad]).reshape(E_PAD // K, K)
    x_p = jnp.pad(x, ((0, N_PAD - N), (0, 0)))

    _sc_deg, _sc_scatter = _build_sc_kernels()
    deg16 = _sc_deg(dst_p)
    dis, xp0 = _k1(deg16, x_p)

    s1 = _sc_scatter(jnp.tile(xp0[None], (2, 1, 1)), src_p, dst_p)
    x1, xp1 = _k3(dis, s1, xp0, xp0, W1, b1.reshape(1, D), W2)

    s2 = _sc_scatter(jnp.tile(xp1[None], (2, 1, 1)), src_p, dst_p)
    x2, xp2 = _k5(dis, s2, xp1, x1, W1, b2.reshape(1, D), W3)

    s3 = _sc_scatter(jnp.tile(xp2[None], (2, 1, 1)), src_p, dst_p)
    out = _k7(dis, s3, xp2, x2, b3.reshape(1, D), Wl, bl.reshape(1, C))
    return out


# speedup vs baseline: 8.2607x; 1.0277x over previous
import functools

import jax
import jax.numpy as jnp
from jax import lax
from jax.experimental import pallas as pl
from jax.experimental.pallas import tpu as pltpu
from jax.experimental.pallas import tpu_sc as plsc

N = 10000
D = 128
C = 64
E = 320000

NC = 2
NS = 16
NW = NC * NS

N_PAD = 10240
K = 128
EPT = 10240
E_PAD = EPT * NW
CHUNKS = EPT // K
ROWS_PT = N_PAD // NS
RB = ROWS_PT // K

CH0 = 120
CH1 = 40

@functools.cache
def _build_sc_kernels():
    mesh = plsc.VectorSubcoreMesh(
        core_axis_name="c", subcore_axis_name="s",
        num_cores=NC, num_subcores=NS)
    sc_deg = functools.partial(
        pl.kernel,
        out_type=jax.ShapeDtypeStruct((NC, N_PAD, 16), jnp.float32),
        mesh=mesh,
        scratch_types=[
            pltpu.VMEM((K,), jnp.int32),
            pltpu.VMEM((K,), jnp.int32),
            pltpu.VMEM((K, 16), jnp.float32),
            pltpu.VMEM((K, 16), jnp.float32),
            pltpu.VMEM_SHARED((N_PAD, 16), jnp.float32),
            pltpu.SemaphoreType.DMA,
            pltpu.SemaphoreType.DMA,
        ],
    )(_sc_deg_body)
    sc_scatter = functools.partial(
        pl.kernel,
        out_type=jax.ShapeDtypeStruct((NC, N_PAD, D), jnp.float32),
        mesh=mesh,
        scratch_types=[
            pltpu.VMEM((CH0, K), jnp.int32),
            pltpu.VMEM((K,), jnp.int32),
            pltpu.VMEM((K,), jnp.int32),
            pltpu.VMEM((K, D), jnp.float32),
            pltpu.VMEM((K, D), jnp.float32),
            pltpu.VMEM_SHARED((N_PAD, D), jnp.float32),
            pltpu.SemaphoreType.DMA,
            pltpu.SemaphoreType.DMA,
        ],
    )(_sc_scatter_body)
    return sc_deg, sc_scatter


def _sc_deg_body(dst_hbm, out_hbm, didx0, didx1, ones_v, stage_v, acc, sem0, sem1):
    cid = lax.axis_index("c")
    sid = lax.axis_index("s")
    crow0 = (cid * NS + sid) * CHUNKS

    def fill(i, _):
        ones_v[i] = jnp.full((16,), 1.0, jnp.float32)
        stage_v[i] = jnp.zeros((16,), jnp.float32)
        return 0

    lax.fori_loop(0, K, fill, 0)
    r0 = sid * ROWS_PT
    for b in range(RB):
        pltpu.sync_copy(stage_v, acc.at[pl.ds(r0 + b * K, K)])
    pltpu.async_copy(dst_hbm.at[crow0], didx0, sem0)
    pltpu.async_copy(dst_hbm.at[crow0 + 1], didx1, sem1)
    plsc.subcore_barrier()

    bufs = ((didx0, sem0), (didx1, sem1))

    def body(j, _):
        for b in range(2):
            didx, sem = bufs[b]
            i = 2 * j + b
            pltpu.make_async_copy(dst_hbm.at[crow0 + i], didx, sem).wait()
            pltpu.sync_copy(ones_v, acc.at[didx], add=True)

            @pl.when(i + 2 < CHUNKS)
            def _():
                pltpu.async_copy(dst_hbm.at[crow0 + i + 2], didx, sem)
        return 0

    lax.fori_loop(0, CHUNKS // 2, body, 0)
    plsc.subcore_barrier()
    for b in range(RB):
        pltpu.sync_copy(acc.at[pl.ds(r0 + b * K, K)], stage_v)
        pltpu.sync_copy(stage_v, out_hbm.at[cid, pl.ds(r0 + b * K, K), :])


def _sc_scatter_body(xp_hbm, src_hbm, dst_hbm, out_hbm,
                     sall, didx0, didx1, rows0, rows1, acc, sem0, sem1):
    cid = lax.axis_index("c")
    sid = lax.axis_index("s")
    crow0 = jnp.where(cid == 0, sid * CH0, NS * CH0 + sid * CH1)
    nch = jnp.where(cid == 0, CH0, CH1)
    tbl = xp_hbm.at[cid]

    def zero(i, _):
        rows0[i // 8, pl.ds((i % 8) * 16, 16)] = jnp.zeros((16,), jnp.float32)
        return 0

    lax.fori_loop(0, K * 8, zero, 0)
    r0 = sid * ROWS_PT
    for b in range(RB):
        pltpu.sync_copy(rows0, acc.at[pl.ds(r0 + b * K, K)])
    @pl.when(cid == 0)
    def _():
        pltpu.sync_copy(src_hbm.at[pl.ds(crow0, CH0)], sall)

    @pl.when(cid == 1)
    def _():
        pltpu.sync_copy(src_hbm.at[pl.ds(crow0, CH1)], sall.at[pl.ds(0, CH1)])

    pltpu.async_copy(dst_hbm.at[crow0], didx0, sem0)
    pltpu.async_copy(tbl.at[sall.at[0]], rows0, sem0)
    pltpu.async_copy(dst_hbm.at[crow0 + 1], didx1, sem1)
    pltpu.async_copy(tbl.at[sall.at[1]], rows1, sem1)
    plsc.subcore_barrier()

    bufs = ((didx0, rows0, sem0), (didx1, rows1, sem1))

    def body(j, _):
        for b in range(2):
            didx, rows, sem = bufs[b]
            i = 2 * j + b
            pltpu.make_async_copy(dst_hbm.at[crow0 + i], didx, sem).wait()
            pltpu.make_async_copy(tbl.at[sall.at[i]], rows, sem).wait()
            pltpu.sync_copy(rows, acc.at[didx], add=True)

            @pl.when(i + 2 < nch)
            def _():
                pltpu.async_copy(dst_hbm.at[crow0 + i + 2], didx, sem)
                pltpu.async_copy(tbl.at[sall.at[i + 2]], rows, sem)
        return 0

    lax.fori_loop(0, nch // 2, body, 0)
    plsc.subcore_barrier()
    for b in range(RB):
        pltpu.sync_copy(acc.at[pl.ds(r0 + b * K, K)], rows0)
        pltpu.sync_copy(rows0, out_hbm.at[cid, pl.ds(r0 + b * K, K), :])


_BT = 256
_GRID = N_PAD // _BT


def _k1_body(deg_ref, x_ref, dis_ref, xp0_ref):
    deg = jnp.sum(deg_ref[...], axis=(0, 2)) + 1.0
    dis = lax.rsqrt(deg)[:, None]
    dis_ref[...] = dis
    xp0_ref[...] = x_ref[...] * dis


def _mid_body(dis_ref, s_ref, xp_ref, xres_ref, w_ref, b_ref, wn_ref,
              xo_ref, xpn_ref, *, first):
    dis = dis_ref[...]
    t = (s_ref[0] + s_ref[1] + xp_ref[...]) * dis
    if first:
        h = jnp.dot(t, w_ref[...], preferred_element_type=jnp.float32)
        xo = jnp.maximum(h + b_ref[...], 0.0)
    else:
        xo = jnp.maximum(xres_ref[...] + t + b_ref[...], 0.0)
    xo_ref[...] = xo
    xpn_ref[...] = jnp.dot(xo, wn_ref[...], preferred_element_type=jnp.float32) * dis


def _k7_body(dis_ref, s_ref, xp_ref, x2_ref, b3_ref, wl_ref, bl_ref, out_ref):
    dis = dis_ref[...]
    t = (s_ref[0] + s_ref[1] + xp_ref[...]) * dis
    x3 = jnp.maximum(x2_ref[...] + t + b3_ref[...], 0.0)
    o = jnp.dot(x3, wl_ref[...], preferred_element_type=jnp.float32) + bl_ref[...]
    m = jnp.max(o, axis=1, keepdims=True)
    lse = jnp.log(jnp.sum(jnp.exp(o - m), axis=1, keepdims=True)) + m
    out_ref[...] = o - lse


def _row_spec(width):
    return pl.BlockSpec((_BT, width), lambda i: (i, 0))


def _full_spec(shape):
    nd = len(shape)
    return pl.BlockSpec(shape, lambda i: (0,) * nd)


_S_SPEC = pl.BlockSpec((NC, _BT, D), lambda i: (0, i, 0))

_k1 = pl.pallas_call(
    _k1_body,
    grid=(_GRID,),
    in_specs=[pl.BlockSpec((NC, _BT, 16), lambda i: (0, i, 0)), _row_spec(D)],
    out_specs=[_row_spec(1), _row_spec(D)],
    out_shape=[
        jax.ShapeDtypeStruct((N_PAD, 1), jnp.float32),
        jax.ShapeDtypeStruct((N_PAD, D), jnp.float32),
    ],
)

_mid_specs = dict(
    grid=(_GRID,),
    in_specs=[
        _row_spec(1), _S_SPEC, _row_spec(D), _row_spec(D),
        _full_spec((D, D)), _full_spec((1, D)), _full_spec((D, D)),
    ],
    out_specs=[_row_spec(D), _row_spec(D)],
    out_shape=[
        jax.ShapeDtypeStruct((N_PAD, D), jnp.float32),
        jax.ShapeDtypeStruct((N_PAD, D), jnp.float32),
    ],
)
_k3 = pl.pallas_call(functools.partial(_mid_body, first=True), **_mid_specs)
_k5 = pl.pallas_call(functools.partial(_mid_body, first=False), **_mid_specs)

_B7 = 80
_k7 = pl.pallas_call(
    _k7_body,
    grid=(N // _B7,),
    in_specs=[
        pl.BlockSpec((_B7, 1), lambda i: (i, 0)),
        pl.BlockSpec((NC, _B7, D), lambda i: (0, i, 0)),
        pl.BlockSpec((_B7, D), lambda i: (i, 0)),
        pl.BlockSpec((_B7, D), lambda i: (i, 0)),
        _full_spec((1, D)), _full_spec((D, C)), _full_spec((1, C)),
    ],
    out_specs=pl.BlockSpec((_B7, C), lambda i: (i, 0)),
    out_shape=jax.ShapeDtypeStruct((N, C), jnp.float32),
)


def kernel(x, edge_index, W1, b1, W2, b2, W3, b3, Wl, bl):
    src = edge_index[0]
    dst = edge_index[1]
    pad = jnp.full((E_PAD - E,), N, jnp.int32)
    src_p = jnp.concatenate([src, pad]).reshape(E_PAD // K, K)
    dst_p = jnp.concatenate([dst, pad]).reshape(E_PAD // K, K)
    x_p = jnp.pad(x, ((0, N_PAD - N), (0, 0)))

    _sc_deg, _sc_scatter = _build_sc_kernels()
    deg16 = _sc_deg(dst_p)
    dis, xp0 = _k1(deg16, x_p)

    s1 = _sc_scatter(jnp.tile(xp0[None], (2, 1, 1)), src_p, dst_p)
    x1, xp1 = _k3(dis, s1, xp0, xp0, W1, b1.reshape(1, D), W2)

    s2 = _sc_scatter(jnp.tile(xp1[None], (2, 1, 1)), src_p, dst_p)
    x2, xp2 = _k5(dis, s2, xp1, x1, W1, b2.reshape(1, D), W3)

    s3 = _sc_scatter(jnp.tile(xp2[None], (2, 1, 1)), src_p, dst_p)
    out = _k7(dis, s3, xp2, x2, b3.reshape(1, D), Wl, bl.reshape(1, C))
    return out


# speedup vs baseline: 20.4092x; 2.4706x over previous
import functools

import jax
import jax.numpy as jnp
from jax import lax
from jax.experimental import pallas as pl
from jax.experimental.pallas import tpu as pltpu
from jax.experimental.pallas import tpu_sc as plsc

N = 10000
D = 128
C = 64
E = 320000

NC = 2
NS = 16
NW = NC * NS

N_PAD = 10240
K = 128
EPT = 10240
E_PAD = EPT * NW
CHUNKS = EPT // K
ROWS_PT = N_PAD // NS
RB = ROWS_PT // K

CH0 = 80
CH1 = 80

@functools.cache
def _build_sc_kernels():
    mesh = plsc.VectorSubcoreMesh(
        core_axis_name="c", subcore_axis_name="s",
        num_cores=NC, num_subcores=NS)
    sc_deg = functools.partial(
        pl.kernel,
        out_type=jax.ShapeDtypeStruct((NC, N_PAD, 16), jnp.float32),
        mesh=mesh,
        scratch_types=[
            pltpu.VMEM((K,), jnp.int32),
            pltpu.VMEM((K,), jnp.int32),
            pltpu.VMEM((K, 16), jnp.float32),
            pltpu.VMEM((K, 16), jnp.float32),
            pltpu.VMEM_SHARED((N_PAD, 16), jnp.float32),
            pltpu.SemaphoreType.DMA,
            pltpu.SemaphoreType.DMA,
        ],
    )(_sc_deg_body)
    sc_scatter = functools.partial(
        pl.kernel,
        out_type=jax.ShapeDtypeStruct((NC, N_PAD, D), jnp.float32),
        mesh=mesh,
        scratch_types=[
            pltpu.VMEM((CH0, K), jnp.int32),
            pltpu.VMEM((K,), jnp.int32),
            pltpu.VMEM((K,), jnp.int32),
            pltpu.VMEM((K, D), jnp.float32),
            pltpu.VMEM((K, D), jnp.float32),
            pltpu.VMEM_SHARED((N_PAD, D), jnp.float32),
            pltpu.SemaphoreType.DMA,
            pltpu.SemaphoreType.DMA,
        ],
    )(_sc_scatter_body)
    return sc_deg, sc_scatter


def _sc_deg_body(dst_hbm, out_hbm, didx0, didx1, ones_v, stage_v, acc, sem0, sem1):
    cid = lax.axis_index("c")
    sid = lax.axis_index("s")
    crow0 = (cid * NS + sid) * CHUNKS

    def fill(i, _):
        ones_v[i] = jnp.full((16,), 1.0, jnp.float32)
        stage_v[i] = jnp.zeros((16,), jnp.float32)
        return 0

    lax.fori_loop(0, K, fill, 0)
    r0 = sid * ROWS_PT
    for b in range(RB):
        pltpu.sync_copy(stage_v, acc.at[pl.ds(r0 + b * K, K)])
    pltpu.async_copy(dst_hbm.at[crow0], didx0, sem0)
    pltpu.async_copy(dst_hbm.at[crow0 + 1], didx1, sem1)
    plsc.subcore_barrier()

    bufs = ((didx0, sem0), (didx1, sem1))

    def body(j, _):
        for b in range(2):
            didx, sem = bufs[b]
            i = 2 * j + b
            pltpu.make_async_copy(dst_hbm.at[crow0 + i], didx, sem).wait()
            pltpu.sync_copy(ones_v, acc.at[didx], add=True)

            @pl.when(i + 2 < CHUNKS)
            def _():
                pltpu.async_copy(dst_hbm.at[crow0 + i + 2], didx, sem)
        return 0

    lax.fori_loop(0, CHUNKS // 2, body, 0)
    plsc.subcore_barrier()
    for b in range(RB):
        pltpu.sync_copy(acc.at[pl.ds(r0 + b * K, K)], stage_v)
        pltpu.sync_copy(stage_v, out_hbm.at[cid, pl.ds(r0 + b * K, K), :])


def _sc_scatter_body(xp_hbm, src_hbm, dst_hbm, out_hbm,
                     sall, didx0, didx1, rows0, rows1, acc, sem0, sem1):
    cid = lax.axis_index("c")
    sid = lax.axis_index("s")
    crow0 = jnp.where(cid == 0, sid * CH0, NS * CH0 + sid * CH1)
    nch = jnp.where(cid == 0, CH0, CH1)
    tbl = xp_hbm.at[cid]

    def zero(i, _):
        rows0[i // 8, pl.ds((i % 8) * 16, 16)] = jnp.zeros((16,), jnp.float32)
        return 0

    lax.fori_loop(0, K * 8, zero, 0)
    r0 = sid * ROWS_PT
    for b in range(RB):
        pltpu.sync_copy(rows0, acc.at[pl.ds(r0 + b * K, K)])
    @pl.when(cid == 0)
    def _():
        pltpu.sync_copy(src_hbm.at[pl.ds(crow0, CH0)], sall)

    @pl.when(cid == 1)
    def _():
        pltpu.sync_copy(src_hbm.at[pl.ds(crow0, CH1)], sall.at[pl.ds(0, CH1)])

    pltpu.async_copy(dst_hbm.at[crow0], didx0, sem0)
    pltpu.async_copy(tbl.at[sall.at[0]], rows0, sem0)
    pltpu.async_copy(dst_hbm.at[crow0 + 1], didx1, sem1)
    pltpu.async_copy(tbl.at[sall.at[1]], rows1, sem1)
    plsc.subcore_barrier()

    bufs = ((didx0, rows0, sem0), (didx1, rows1, sem1))

    def body(j, _):
        for b in range(2):
            didx, rows, sem = bufs[b]
            i = 2 * j + b
            pltpu.make_async_copy(dst_hbm.at[crow0 + i], didx, sem).wait()
            pltpu.make_async_copy(tbl.at[sall.at[i]], rows, sem).wait()
            pltpu.sync_copy(rows, acc.at[didx], add=True)

            @pl.when(i + 2 < nch)
            def _():
                pltpu.async_copy(dst_hbm.at[crow0 + i + 2], didx, sem)
                pltpu.async_copy(tbl.at[sall.at[i + 2]], rows, sem)
        return 0

    lax.fori_loop(0, nch // 2, body, 0)
    plsc.subcore_barrier()
    for b in range(RB):
        pltpu.sync_copy(acc.at[pl.ds(r0 + b * K, K)], rows0)
        pltpu.sync_copy(rows0, out_hbm.at[cid, pl.ds(r0 + b * K, K), :])


_BT = 256
_GRID = N_PAD // _BT


def _k1_body(deg_ref, x_ref, dis_ref, xp0_ref):
    deg = jnp.sum(deg_ref[...], axis=(0, 2)) + 1.0
    dis = lax.rsqrt(deg)[:, None]
    dis_ref[...] = dis
    xp0_ref[...] = x_ref[...] * dis


def _mid_body(dis_ref, s_ref, xp_ref, xres_ref, w_ref, b_ref, wn_ref,
              xo_ref, xpn_ref, *, first):
    dis = dis_ref[...]
    t = (s_ref[0] + s_ref[1] + xp_ref[...]) * dis
    if first:
        h = jnp.dot(t, w_ref[...], preferred_element_type=jnp.float32)
        xo = jnp.maximum(h + b_ref[...], 0.0)
    else:
        xo = jnp.maximum(xres_ref[...] + t + b_ref[...], 0.0)
    xo_ref[...] = xo
    xpn_ref[...] = jnp.dot(xo, wn_ref[...], preferred_element_type=jnp.float32) * dis


def _k7_body(dis_ref, s_ref, xp_ref, x2_ref, b3_ref, wl_ref, bl_ref, out_ref):
    dis = dis_ref[...]
    t = (s_ref[0] + s_ref[1] + xp_ref[...]) * dis
    x3 = jnp.maximum(x2_ref[...] + t + b3_ref[...], 0.0)
    o = jnp.dot(x3, wl_ref[...], preferred_element_type=jnp.float32) + bl_ref[...]
    m = jnp.max(o, axis=1, keepdims=True)
    lse = jnp.log(jnp.sum(jnp.exp(o - m), axis=1, keepdims=True)) + m
    out_ref[...] = o - lse


def _row_spec(width):
    return pl.BlockSpec((_BT, width), lambda i: (i, 0))


def _full_spec(shape):
    nd = len(shape)
    return pl.BlockSpec(shape, lambda i: (0,) * nd)


_S_SPEC = pl.BlockSpec((NC, _BT, D), lambda i: (0, i, 0))

_k1 = pl.pallas_call(
    _k1_body,
    grid=(_GRID,),
    in_specs=[pl.BlockSpec((NC, _BT, 16), lambda i: (0, i, 0)), _row_spec(D)],
    out_specs=[_row_spec(1), _row_spec(D)],
    out_shape=[
        jax.ShapeDtypeStruct((N_PAD, 1), jnp.float32),
        jax.ShapeDtypeStruct((N_PAD, D), jnp.float32),
    ],
)

_mid_specs = dict(
    grid=(_GRID,),
    in_specs=[
        _row_spec(1), _S_SPEC, _row_spec(D), _row_spec(D),
        _full_spec((D, D)), _full_spec((1, D)), _full_spec((D, D)),
    ],
    out_specs=[_row_spec(D), _row_spec(D)],
    out_shape=[
        jax.ShapeDtypeStruct((N_PAD, D), jnp.float32),
        jax.ShapeDtypeStruct((N_PAD, D), jnp.float32),
    ],
)
_k3 = pl.pallas_call(functools.partial(_mid_body, first=True), **_mid_specs)
_k5 = pl.pallas_call(functools.partial(_mid_body, first=False), **_mid_specs)

_B7 = 80
_k7 = pl.pallas_call(
    _k7_body,
    grid=(N // _B7,),
    in_specs=[
        pl.BlockSpec((_B7, 1), lambda i: (i, 0)),
        pl.BlockSpec((NC, _B7, D), lambda i: (0, i, 0)),
        pl.BlockSpec((_B7, D), lambda i: (i, 0)),
        pl.BlockSpec((_B7, D), lambda i: (i, 0)),
        _full_spec((1, D)), _full_spec((D, C)), _full_spec((1, C)),
    ],
    out_specs=pl.BlockSpec((_B7, C), lambda i: (i, 0)),
    out_shape=jax.ShapeDtypeStruct((N, C), jnp.float32),
)


def kernel(x, edge_index, W1, b1, W2, b2, W3, b3, Wl, bl):
    src = edge_index[0]
    dst = edge_index[1]
    pad = N + jnp.arange(E_PAD - E, dtype=jnp.int32) % (N_PAD - N)
    src_p = jnp.concatenate([src, pad]).reshape(E_PAD // K, K)
    dst_p = jnp.concatenate([dst, pad]).reshape(E_PAD // K, K)
    x_p = jnp.pad(x, ((0, N_PAD - N), (0, 0)))

    _sc_deg, _sc_scatter = _build_sc_kernels()
    deg16 = _sc_deg(dst_p)
    dis, xp0 = _k1(deg16, x_p)

    s1 = _sc_scatter(jnp.tile(xp0[None], (2, 1, 1)), src_p, dst_p)
    x1, xp1 = _k3(dis, s1, xp0, xp0, W1, b1.reshape(1, D), W2)

    s2 = _sc_scatter(jnp.tile(xp1[None], (2, 1, 1)), src_p, dst_p)
    x2, xp2 = _k5(dis, s2, xp1, x1, W1, b2.reshape(1, D), W3)

    s3 = _sc_scatter(jnp.tile(xp2[None], (2, 1, 1)), src_p, dst_p)
    out = _k7(dis, s3, xp2, x2, b3.reshape(1, D), Wl, bl.reshape(1, C))
    return out


# speedup vs baseline: 21.0125x; 1.0296x over previous
import functools

import jax
import jax.numpy as jnp
from jax import lax
from jax.experimental import pallas as pl
from jax.experimental.pallas import tpu as pltpu
from jax.experimental.pallas import tpu_sc as plsc

N = 10000
D = 128
C = 64
E = 320000

NC = 2
NS = 16
NW = NC * NS

N_PAD = 10240
K = 128
EPT = 10240
E_PAD = EPT * NW
CHUNKS = EPT // K
ROWS_PT = N_PAD // NS
RB = ROWS_PT // K

CH0 = 80
CH1 = 80

@functools.cache
def _build_sc_kernels():
    mesh = plsc.VectorSubcoreMesh(
        core_axis_name="c", subcore_axis_name="s",
        num_cores=NC, num_subcores=NS)
    sc_deg = functools.partial(
        pl.kernel,
        out_type=jax.ShapeDtypeStruct((NC, N_PAD, 16), jnp.float32),
        mesh=mesh,
        scratch_types=[
            pltpu.VMEM((K,), jnp.int32),
            pltpu.VMEM((K,), jnp.int32),
            pltpu.VMEM((K, 16), jnp.float32),
            pltpu.VMEM((K, 16), jnp.float32),
            pltpu.VMEM_SHARED((N_PAD, 16), jnp.float32),
            pltpu.SemaphoreType.DMA,
            pltpu.SemaphoreType.DMA,
        ],
    )(_sc_deg_body)
    sc_scatter = functools.partial(
        pl.kernel,
        out_type=jax.ShapeDtypeStruct((NC, N_PAD, D), jnp.float32),
        mesh=mesh,
        scratch_types=[
            pltpu.VMEM((CH0, K), jnp.int32),
            pltpu.VMEM((K,), jnp.int32),
            pltpu.VMEM((K,), jnp.int32),
            pltpu.VMEM((K, D), jnp.float32),
            pltpu.VMEM((K, D), jnp.float32),
            pltpu.VMEM_SHARED((N_PAD, D), jnp.float32),
            pltpu.SemaphoreType.DMA,
            pltpu.SemaphoreType.DMA,
        ],
    )(_sc_scatter_body)
    return sc_deg, sc_scatter


def _sc_deg_body(dst_hbm, out_hbm, didx0, didx1, ones_v, stage_v, acc, sem0, sem1):
    cid = lax.axis_index("c")
    sid = lax.axis_index("s")
    crow0 = (cid * NS + sid) * CHUNKS

    def fill(i, _):
        ones_v[i] = jnp.full((16,), 1.0, jnp.float32)
        stage_v[i] = jnp.zeros((16,), jnp.float32)
        return 0

    lax.fori_loop(0, K, fill, 0)
    r0 = sid * ROWS_PT
    for b in range(RB):
        pltpu.sync_copy(stage_v, acc.at[pl.ds(r0 + b * K, K)])
    pltpu.async_copy(dst_hbm.at[crow0], didx0, sem0)
    pltpu.async_copy(dst_hbm.at[crow0 + 1], didx1, sem1)
    plsc.subcore_barrier()

    bufs = ((didx0, sem0), (didx1, sem1))

    def body(j, _):
        for b in range(2):
            didx, sem = bufs[b]
            i = 2 * j + b
            pltpu.make_async_copy(dst_hbm.at[crow0 + i], didx, sem).wait()
            pltpu.sync_copy(ones_v, acc.at[didx], add=True)

            @pl.when(i + 2 < CHUNKS)
            def _():
                pltpu.async_copy(dst_hbm.at[crow0 + i + 2], didx, sem)
        return 0

    lax.fori_loop(0, CHUNKS // 2, body, 0)
    plsc.subcore_barrier()
    for b in range(RB):
        pltpu.sync_copy(acc.at[pl.ds(r0 + b * K, K)], stage_v)
        pltpu.sync_copy(stage_v, out_hbm.at[cid, pl.ds(r0 + b * K, K), :])


def _sc_scatter_body(xp_hbm, src_hbm, dst_hbm, out_hbm,
                     sall, didx0, didx1, rows0, rows1, acc, sem0, sem1):
    cid = lax.axis_index("c")
    sid = lax.axis_index("s")
    crow0 = jnp.where(cid == 0, sid * CH0, NS * CH0 + sid * CH1)
    nch = jnp.where(cid == 0, CH0, CH1)
    tbl = xp_hbm

    def zero(i, _):
        rows0[i // 8, pl.ds((i % 8) * 16, 16)] = jnp.zeros((16,), jnp.float32)
        return 0

    lax.fori_loop(0, K * 8, zero, 0)
    r0 = sid * ROWS_PT
    for b in range(RB):
        pltpu.sync_copy(rows0, acc.at[pl.ds(r0 + b * K, K)])
    @pl.when(cid == 0)
    def _():
        pltpu.sync_copy(src_hbm.at[pl.ds(crow0, CH0)], sall)

    @pl.when(cid == 1)
    def _():
        pltpu.sync_copy(src_hbm.at[pl.ds(crow0, CH1)], sall.at[pl.ds(0, CH1)])

    pltpu.async_copy(dst_hbm.at[crow0], didx0, sem0)
    pltpu.async_copy(tbl.at[sall.at[0]], rows0, sem0)
    pltpu.async_copy(dst_hbm.at[crow0 + 1], didx1, sem1)
    pltpu.async_copy(tbl.at[sall.at[1]], rows1, sem1)
    plsc.subcore_barrier()

    bufs = ((didx0, rows0, sem0), (didx1, rows1, sem1))

    def body(j, _):
        for b in range(2):
            didx, rows, sem = bufs[b]
            i = 2 * j + b
            pltpu.make_async_copy(dst_hbm.at[crow0 + i], didx, sem).wait()
            pltpu.make_async_copy(tbl.at[sall.at[i]], rows, sem).wait()
            pltpu.sync_copy(rows, acc.at[didx], add=True)

            @pl.when(i + 2 < nch)
            def _():
                pltpu.async_copy(dst_hbm.at[crow0 + i + 2], didx, sem)
                pltpu.async_copy(tbl.at[sall.at[i + 2]], rows, sem)
        return 0

    lax.fori_loop(0, nch // 2, body, 0)
    plsc.subcore_barrier()
    for b in range(RB):
        pltpu.sync_copy(acc.at[pl.ds(r0 + b * K, K)], rows0)
        pltpu.sync_copy(rows0, out_hbm.at[cid, pl.ds(r0 + b * K, K), :])


_BT = 256
_GRID = N_PAD // _BT


def _k1_body(deg_ref, x_ref, dis_ref, xp0_ref):
    deg = jnp.sum(deg_ref[...], axis=(0, 2)) + 1.0
    dis = lax.rsqrt(deg)[:, None]
    dis_ref[...] = dis
    xp0_ref[...] = x_ref[...] * dis


def _mid_body(dis_ref, s_ref, xp_ref, xres_ref, w_ref, b_ref, wn_ref,
              xo_ref, xpn_ref, *, first):
    dis = dis_ref[...]
    t = (s_ref[0] + s_ref[1] + xp_ref[...]) * dis
    if first:
        h = jnp.dot(t, w_ref[...], preferred_element_type=jnp.float32)
        xo = jnp.maximum(h + b_ref[...], 0.0)
    else:
        xo = jnp.maximum(xres_ref[...] + t + b_ref[...], 0.0)
    xo_ref[...] = xo
    xpn_ref[...] = jnp.dot(xo, wn_ref[...], preferred_element_type=jnp.float32) * dis


def _k7_body(dis_ref, s_ref, xp_ref, x2_ref, b3_ref, wl_ref, bl_ref, out_ref):
    dis = dis_ref[...]
    t = (s_ref[0] + s_ref[1] + xp_ref[...]) * dis
    x3 = jnp.maximum(x2_ref[...] + t + b3_ref[...], 0.0)
    o = jnp.dot(x3, wl_ref[...], preferred_element_type=jnp.float32) + bl_ref[...]
    m = jnp.max(o, axis=1, keepdims=True)
    lse = jnp.log(jnp.sum(jnp.exp(o - m), axis=1, keepdims=True)) + m
    out_ref[...] = o - lse


def _row_spec(width):
    return pl.BlockSpec((_BT, width), lambda i: (i, 0))


def _full_spec(shape):
    nd = len(shape)
    return pl.BlockSpec(shape, lambda i: (0,) * nd)


_S_SPEC = pl.BlockSpec((NC, _BT, D), lambda i: (0, i, 0))

_k1 = pl.pallas_call(
    _k1_body,
    grid=(_GRID,),
    in_specs=[pl.BlockSpec((NC, _BT, 16), lambda i: (0, i, 0)), _row_spec(D)],
    out_specs=[_row_spec(1), _row_spec(D)],
    out_shape=[
        jax.ShapeDtypeStruct((N_PAD, 1), jnp.float32),
        jax.ShapeDtypeStruct((N_PAD, D), jnp.float32),
    ],
)

_mid_specs = dict(
    grid=(_GRID,),
    in_specs=[
        _row_spec(1), _S_SPEC, _row_spec(D), _row_spec(D),
        _full_spec((D, D)), _full_spec((1, D)), _full_spec((D, D)),
    ],
    out_specs=[_row_spec(D), _row_spec(D)],
    out_shape=[
        jax.ShapeDtypeStruct((N_PAD, D), jnp.float32),
        jax.ShapeDtypeStruct((N_PAD, D), jnp.float32),
    ],
)
_k3 = pl.pallas_call(functools.partial(_mid_body, first=True), **_mid_specs)
_k5 = pl.pallas_call(functools.partial(_mid_body, first=False), **_mid_specs)

_B7 = 80
_k7 = pl.pallas_call(
    _k7_body,
    grid=(N // _B7,),
    in_specs=[
        pl.BlockSpec((_B7, 1), lambda i: (i, 0)),
        pl.BlockSpec((NC, _B7, D), lambda i: (0, i, 0)),
        pl.BlockSpec((_B7, D), lambda i: (i, 0)),
        pl.BlockSpec((_B7, D), lambda i: (i, 0)),
        _full_spec((1, D)), _full_spec((D, C)), _full_spec((1, C)),
    ],
    out_specs=pl.BlockSpec((_B7, C), lambda i: (i, 0)),
    out_shape=jax.ShapeDtypeStruct((N, C), jnp.float32),
)


def kernel(x, edge_index, W1, b1, W2, b2, W3, b3, Wl, bl):
    src = edge_index[0]
    dst = edge_index[1]
    pad = N + jnp.arange(E_PAD - E, dtype=jnp.int32) % (N_PAD - N)
    src_p = jnp.concatenate([src, pad]).reshape(E_PAD // K, K)
    dst_p = jnp.concatenate([dst, pad]).reshape(E_PAD // K, K)
    x_p = jnp.pad(x, ((0, N_PAD - N), (0, 0)))

    _sc_deg, _sc_scatter = _build_sc_kernels()
    deg16 = _sc_deg(dst_p)
    dis, xp0 = _k1(deg16, x_p)

    s1 = _sc_scatter(xp0, src_p, dst_p)
    x1, xp1 = _k3(dis, s1, xp0, xp0, W1, b1.reshape(1, D), W2)

    s2 = _sc_scatter(xp1, src_p, dst_p)
    x2, xp2 = _k5(dis, s2, xp1, x1, W1, b2.reshape(1, D), W3)

    s3 = _sc_scatter(xp2, src_p, dst_p)
    out = _k7(dis, s3, xp2, x2, b3.reshape(1, D), Wl, bl.reshape(1, C))
    return out


# speedup vs baseline: 26.1282x; 1.2435x over previous
import functools

import jax
import jax.numpy as jnp
from jax import lax
from jax.experimental import pallas as pl
from jax.experimental.pallas import tpu as pltpu
from jax.experimental.pallas import tpu_sc as plsc

N = 10000
D = 128
C = 64
E = 320000

NC = 2
NS = 16
NW = NC * NS

N_PAD = 10240
K = 128
EPT = 10240
E_PAD = EPT * NW
CHUNKS = EPT // K
ROWS_PT = N_PAD // NS
RB = ROWS_PT // K

CH0 = 80
CH1 = 80

@functools.cache
def _build_sc_kernels():
    mesh = plsc.VectorSubcoreMesh(
        core_axis_name="c", subcore_axis_name="s",
        num_cores=NC, num_subcores=NS)
    sc_deg = functools.partial(
        pl.kernel,
        out_type=jax.ShapeDtypeStruct((NC, N_PAD, 16), jnp.float32),
        mesh=mesh,
        scratch_types=[
            pltpu.VMEM((K,), jnp.int32),
            pltpu.VMEM((K,), jnp.int32),
            pltpu.VMEM((K, 16), jnp.float32),
            pltpu.VMEM((K, 16), jnp.float32),
            pltpu.VMEM_SHARED((N_PAD, 16), jnp.float32),
            pltpu.SemaphoreType.DMA,
            pltpu.SemaphoreType.DMA,
        ],
    )(_sc_deg_body)
    sc_scatter = functools.partial(
        pl.kernel,
        out_type=jax.ShapeDtypeStruct((NC, N_PAD, D), jnp.float32),
        mesh=mesh,
        scratch_types=[
            pltpu.VMEM((CH0, K), jnp.int32),
            pltpu.VMEM((K,), jnp.int32),
            pltpu.VMEM((K,), jnp.int32),
            pltpu.VMEM((K, D), jnp.float32),
            pltpu.VMEM((K, D), jnp.float32),
            pltpu.VMEM_SHARED((N_PAD, D), jnp.float32),
            pltpu.SemaphoreType.DMA,
            pltpu.SemaphoreType.DMA,
        ],
    )(_sc_scatter_body)
    return sc_deg, sc_scatter


def _sc_deg_body(dst_hbm, out_hbm, didx0, didx1, ones_v, stage_v, acc, sem0, sem1):
    cid = lax.axis_index("c")
    sid = lax.axis_index("s")
    crow0 = (cid * NS + sid) * CHUNKS

    def fill(i, _):
        ones_v[i] = jnp.full((16,), 1.0, jnp.float32)
        stage_v[i] = jnp.zeros((16,), jnp.float32)
        return 0

    lax.fori_loop(0, K, fill, 0)
    r0 = sid * ROWS_PT
    for b in range(RB):
        pltpu.sync_copy(stage_v, acc.at[pl.ds(r0 + b * K, K)])
    pltpu.async_copy(dst_hbm.at[crow0], didx0, sem0)
    pltpu.async_copy(dst_hbm.at[crow0 + 1], didx1, sem1)
    plsc.subcore_barrier()

    bufs = ((didx0, sem0), (didx1, sem1))

    def body(j, _):
        for b in range(2):
            didx, sem = bufs[b]
            i = 2 * j + b
            pltpu.make_async_copy(dst_hbm.at[crow0 + i], didx, sem).wait()
            pltpu.sync_copy(ones_v, acc.at[didx], add=True)

            @pl.when(i + 2 < CHUNKS)
            def _():
                pltpu.async_copy(dst_hbm.at[crow0 + i + 2], didx, sem)
        return 0

    lax.fori_loop(0, CHUNKS // 2, body, 0)
    plsc.subcore_barrier()
    for b in range(RB):
        pltpu.sync_copy(acc.at[pl.ds(r0 + b * K, K)], stage_v)
        pltpu.sync_copy(stage_v, out_hbm.at[cid, pl.ds(r0 + b * K, K), :])


def _sc_scatter_body(xp_hbm, src_hbm, dst_hbm, out_hbm,
                     sall, didx0, didx1, rows0, rows1, acc, sem0, sem1):
    cid = lax.axis_index("c")
    sid = lax.axis_index("s")
    crow0 = jnp.where(cid == 0, sid * CH0, NS * CH0 + sid * CH1)
    nch = jnp.where(cid == 0, CH0, CH1)
    tbl = xp_hbm

    def zero(i, _):
        rows0[i // 8, pl.ds((i % 8) * 16, 16)] = jnp.zeros((16,), jnp.float32)
        return 0

    lax.fori_loop(0, K * 8, zero, 0)
    r0 = sid * ROWS_PT
    for b in range(RB):
        pltpu.sync_copy(rows0, acc.at[pl.ds(r0 + b * K, K)])
    @pl.when(cid == 0)
    def _():
        pltpu.sync_copy(src_hbm.at[pl.ds(crow0, CH0)], sall)

    @pl.when(cid == 1)
    def _():
        pltpu.sync_copy(src_hbm.at[pl.ds(crow0, CH1)], sall.at[pl.ds(0, CH1)])

    pltpu.async_copy(dst_hbm.at[crow0], didx0, sem0)
    pltpu.async_copy(tbl.at[sall.at[0]], rows0, sem0)
    pltpu.async_copy(dst_hbm.at[crow0 + 1], didx1, sem1)
    pltpu.async_copy(tbl.at[sall.at[1]], rows1, sem1)
    plsc.subcore_barrier()

    bufs = ((didx0, rows0, sem0), (didx1, rows1, sem1))

    def body(j, _):
        for b in range(2):
            didx, rows, sem = bufs[b]
            i = 2 * j + b
            pltpu.make_async_copy(dst_hbm.at[crow0 + i], didx, sem).wait()
            pltpu.make_async_copy(tbl.at[sall.at[i]], rows, sem).wait()
            pltpu.sync_copy(rows, acc.at[didx], add=True)

            @pl.when(i + 2 < nch)
            def _():
                pltpu.async_copy(dst_hbm.at[crow0 + i + 2], didx, sem)
                pltpu.async_copy(tbl.at[sall.at[i + 2]], rows, sem)
        return 0

    lax.fori_loop(0, nch // 2, body, 0)
    plsc.subcore_barrier()
    for b in range(RB):
        pltpu.sync_copy(acc.at[pl.ds(r0 + b * K, K)], rows0)
        pltpu.sync_copy(rows0, out_hbm.at[cid, pl.ds(r0 + b * K, K), :])


_BT = 1024
_GRID = N_PAD // _BT


def _k1_body(deg_ref, x_ref, dis_ref, xp0_ref):
    deg = jnp.sum(deg_ref[...], axis=(0, 2)) + 1.0
    dis = lax.rsqrt(deg)[:, None]
    dis_ref[...] = dis
    xp0_ref[...] = x_ref[...] * dis


def _mid_body(dis_ref, s_ref, xp_ref, xres_ref, w_ref, b_ref, wn_ref,
              xo_ref, xpn_ref, *, first):
    dis = dis_ref[...]
    t = (s_ref[0] + s_ref[1] + xp_ref[...]) * dis
    if first:
        h = jnp.dot(t, w_ref[...], preferred_element_type=jnp.float32)
        xo = jnp.maximum(h + b_ref[...], 0.0)
    else:
        xo = jnp.maximum(xres_ref[...] + t + b_ref[...], 0.0)
    xo_ref[...] = xo
    xpn_ref[...] = jnp.dot(xo, wn_ref[...], preferred_element_type=jnp.float32) * dis


def _k7_body(dis_ref, s_ref, xp_ref, x2_ref, b3_ref, wl_ref, bl_ref, out_ref):
    dis = dis_ref[...]
    t = (s_ref[0] + s_ref[1] + xp_ref[...]) * dis
    x3 = jnp.maximum(x2_ref[...] + t + b3_ref[...], 0.0)
    o = jnp.dot(x3, wl_ref[...], preferred_element_type=jnp.float32) + bl_ref[...]
    m = jnp.max(o, axis=1, keepdims=True)
    lse = jnp.log(jnp.sum(jnp.exp(o - m), axis=1, keepdims=True)) + m
    out_ref[...] = o - lse


def _row_spec(width):
    return pl.BlockSpec((_BT, width), lambda i: (i, 0))


def _full_spec(shape):
    nd = len(shape)
    return pl.BlockSpec(shape, lambda i: (0,) * nd)


_S_SPEC = pl.BlockSpec((NC, _BT, D), lambda i: (0, i, 0))

_k1 = pl.pallas_call(
    _k1_body,
    grid=(_GRID,),
    in_specs=[pl.BlockSpec((NC, _BT, 16), lambda i: (0, i, 0)), _row_spec(D)],
    out_specs=[_row_spec(1), _row_spec(D)],
    out_shape=[
        jax.ShapeDtypeStruct((N_PAD, 1), jnp.float32),
        jax.ShapeDtypeStruct((N_PAD, D), jnp.float32),
    ],
)

_mid_specs = dict(
    grid=(_GRID,),
    in_specs=[
        _row_spec(1), _S_SPEC, _row_spec(D), _row_spec(D),
        _full_spec((D, D)), _full_spec((1, D)), _full_spec((D, D)),
    ],
    out_specs=[_row_spec(D), _row_spec(D)],
    out_shape=[
        jax.ShapeDtypeStruct((N_PAD, D), jnp.float32),
        jax.ShapeDtypeStruct((N_PAD, D), jnp.float32),
    ],
)
_k3 = pl.pallas_call(functools.partial(_mid_body, first=True), **_mid_specs)
_k5 = pl.pallas_call(functools.partial(_mid_body, first=False), **_mid_specs)

_B7 = 400
_k7 = pl.pallas_call(
    _k7_body,
    grid=(N // _B7,),
    in_specs=[
        pl.BlockSpec((_B7, 1), lambda i: (i, 0)),
        pl.BlockSpec((NC, _B7, D), lambda i: (0, i, 0)),
        pl.BlockSpec((_B7, D), lambda i: (i, 0)),
        pl.BlockSpec((_B7, D), lambda i: (i, 0)),
        _full_spec((1, D)), _full_spec((D, C)), _full_spec((1, C)),
    ],
    out_specs=pl.BlockSpec((_B7, C), lambda i: (i, 0)),
    out_shape=jax.ShapeDtypeStruct((N, C), jnp.float32),
)


def kernel(x, edge_index, W1, b1, W2, b2, W3, b3, Wl, bl):
    src = edge_index[0]
    dst = edge_index[1]
    pad = N + jnp.arange(E_PAD - E, dtype=jnp.int32) % (N_PAD - N)
    src_p = jnp.concatenate([src, pad]).reshape(E_PAD // K, K)
    dst_p = jnp.concatenate([dst, pad]).reshape(E_PAD // K, K)
    x_p = jnp.pad(x, ((0, N_PAD - N), (0, 0)))

    _sc_deg, _sc_scatter = _build_sc_kernels()
    deg16 = _sc_deg(dst_p)
    dis, xp0 = _k1(deg16, x_p)

    s1 = _sc_scatter(xp0, src_p, dst_p)
    x1, xp1 = _k3(dis, s1, xp0, xp0, W1, b1.reshape(1, D), W2)

    s2 = _sc_scatter(xp1, src_p, dst_p)
    x2, xp2 = _k5(dis, s2, xp1, x1, W1, b2.reshape(1, D), W3)

    s3 = _sc_scatter(xp2, src_p, dst_p)
    out = _k7(dis, s3, xp2, x2, b3.reshape(1, D), Wl, bl.reshape(1, C))
    return out
